# baseline (speedup 1.0000x reference)
import jax
import jax.numpy as jnp
from jax import lax
from jax.experimental import pallas as pl
from jax.experimental.pallas import tpu as pltpu
from jax.experimental.pallas import tpu_sc as plsc

N = 10000
E = 320000
D = 128
DE = 16
NLAYER = 3

NC, NS, LANES = 2, 16, 16
NW = NC * NS
N_PAD = 10240
CHUNK = 128
EPT = 10240
NCHUNK = EPT // CHUNK
E_PAD = EPT * NW
ROWS_PT = N_PAD // NS

_mesh = plsc.VectorSubcoreMesh(core_axis_name="c", subcore_axis_name="s")


def _hist_body(col_hbm, out_hbm, colv, ones_b, acc):
  c = lax.axis_index("c")
  s = lax.axis_index("s")
  wid = s * NC + c
  pltpu.sync_copy(col_hbm.at[wid], colv)
  onerow = jnp.where(jnp.arange(LANES, dtype=jnp.int32) == 0, 1.0, 0.0)
  zrow = jnp.zeros((LANES,), jnp.float32)

  @pl.loop(0, CHUNK)
  def _(i):
    for b in range(D // LANES):
      ones_b[i, pl.ds(b * LANES, LANES)] = zrow

  for p in range(ROWS_PT // CHUNK):
    pltpu.sync_copy(ones_b, acc.at[pl.ds(s * ROWS_PT + p * CHUNK, CHUNK)])

  @pl.loop(0, CHUNK)
  def _(i):
    ones_b[i, pl.ds(0, LANES)] = onerow

  plsc.subcore_barrier()

  @pl.loop(0, NCHUNK)
  def _(j):
    pltpu.sync_copy(ones_b, acc.at[colv.at[j]], add=True)

  plsc.subcore_barrier()
  for p in range(ROWS_PT // CHUNK):
    base = s * ROWS_PT + p * CHUNK
    pltpu.sync_copy(acc.at[pl.ds(base, CHUNK)], ones_b)
    pltpu.sync_copy(ones_b, out_hbm.at[c, pl.ds(base, CHUNK)])


_hist = pl.kernel(
    _hist_body,
    out_type=jax.ShapeDtypeStruct((NC, N_PAD, D), jnp.float32),
    mesh=_mesh,
    scratch_types=[
        pltpu.VMEM((NCHUNK, CHUNK), jnp.int32),
        pltpu.VMEM((CHUNK, D), jnp.float32),
        pltpu.VMEM_SHARED((N_PAD, D), jnp.float32),
    ],
)


GC = 128
QCS = 16
GCHUNK_TOT = E_PAD // GC
C0G = 128
C1G = (GCHUNK_TOT - 16 * C0G) // 16
GNBUF = 2
ZR = 64


def _scatter_body(m_hbm, row_hbm, col_hbm, out_hbm,
                  rowq, colq, g0, g1, zbuf, acc, s0, s1):
  c = lax.axis_index("c")
  s = lax.axis_index("s")
  bufs = (g0, g1)
  sems = (s0, s1)
  zrow = jnp.zeros((LANES,), jnp.float32)

  gstart = jnp.where(c == 0, s * C0G, 16 * C0G + s * C1G)
  nst = jnp.where(c == 0, C0G // QCS, C1G // QCS)

  pltpu.sync_copy(row_hbm.at[pl.ds(gstart, QCS)], rowq)
  pltpu.sync_copy(col_hbm.at[pl.ds(gstart, QCS)], colq)
  for b in range(GNBUF):
    pltpu.async_copy(m_hbm.at[rowq.at[b]], bufs[b], sems[b])

  @pl.loop(0, ZR)
  def _(i):
    for b in range(D // LANES):
      zbuf[i, pl.ds(b * LANES, LANES)] = zrow

  for p in range(ROWS_PT // ZR):
    pltpu.sync_copy(zbuf, acc.at[pl.ds(s * ROWS_PT + p * ZR, ZR)])
  plsc.subcore_barrier()

  @pl.loop(0, nst)
  def _(st):
    gb = gstart + st * QCS

    @pl.when(st > 0)
    def _():
      pltpu.sync_copy(row_hbm.at[pl.ds(gb, QCS)], rowq)
      pltpu.sync_copy(col_hbm.at[pl.ds(gb, QCS)], colq)
      for b in range(GNBUF):
        pltpu.async_copy(m_hbm.at[rowq.at[b]], bufs[b], sems[b])

    @pl.loop(0, QCS, step=GNBUF)
    def _(j):
      for b in range(GNBUF):
        k = j + b
        pltpu.make_async_copy(m_hbm.at[rowq.at[k]], bufs[b], sems[b]).wait()
        pltpu.sync_copy(bufs[b], acc.at[colq.at[k]], add=True)

        @pl.when(k + GNBUF < QCS)
        def _():
          pltpu.async_copy(m_hbm.at[rowq.at[k + GNBUF]], bufs[b], sems[b])

  plsc.subcore_barrier()
  for p in range(ROWS_PT // GC):
    base = s * ROWS_PT + p * GC
    pltpu.sync_copy(acc.at[pl.ds(base, GC)], out_hbm.at[c, pl.ds(base, GC)])


_scatter = pl.kernel(
    _scatter_body,
    out_type=jax.ShapeDtypeStruct((NC, N_PAD, D), jnp.float32),
    mesh=_mesh,
    scratch_types=[
        pltpu.VMEM((QCS, GC), jnp.int32),
        pltpu.VMEM((QCS, GC), jnp.int32),
        pltpu.VMEM((GC, D), jnp.float32),
        pltpu.VMEM((GC, D), jnp.float32),
        pltpu.VMEM((ZR, D), jnp.float32),
        pltpu.VMEM_SHARED((N_PAD, D), jnp.float32),
        pltpu.SemaphoreType.DMA,
        pltpu.SemaphoreType.DMA,
    ],
)


def _edgeout_body(a_hbm, c_hbm, t_hbm, row_hbm, col_hbm, out_hbm,
                  av, cv, tv, rowv, colv, obuf):
  c = lax.axis_index("c")
  s = lax.axis_index("s")
  wid = s * NC + c
  pltpu.sync_copy(a_hbm, av)
  pltpu.sync_copy(c_hbm, cv)
  pltpu.sync_copy(t_hbm.at[wid], tv)
  pltpu.sync_copy(row_hbm.at[wid], rowv)
  pltpu.sync_copy(col_hbm.at[wid], colv)

  @pl.loop(0, NCHUNK)
  def _(j):
    for b in range(CHUNK // LANES):
      sl = pl.ds(b * LANES, LANES)
      rv = rowv[j, sl]
      cc = colv[j, sl]
      o = plsc.load_gather(av, [rv]) + plsc.load_gather(cv, [cc]) + tv[j, sl]
      obuf[j, sl] = o

  pltpu.sync_copy(obuf, out_hbm.at[wid])


_edgeout = pl.kernel(
    _edgeout_body,
    out_type=jax.ShapeDtypeStruct((NW, NCHUNK, CHUNK), jnp.float32),
    mesh=_mesh,
    compiler_params=pltpu.CompilerParams(needs_layout_passes=False),
    scratch_types=[
        pltpu.VMEM((N_PAD,), jnp.float32),
        pltpu.VMEM((N_PAD,), jnp.float32),
        pltpu.VMEM((NCHUNK, CHUNK), jnp.float32),
        pltpu.VMEM((NCHUNK, CHUNK), jnp.int32),
        pltpu.VMEM((NCHUNK, CHUNK), jnp.int32),
        pltpu.VMEM((NCHUNK, CHUNK), jnp.float32),
    ],
)


BM = 1024
BE = 512


def _dinv_body(dp_ref, out_ref):
  deg = dp_ref[0, :, :1] + dp_ref[1, :, :1] + 1.0
  dv = lax.rsqrt(jnp.maximum(deg, 1e-12))
  out_ref[...] = jnp.broadcast_to(dv, (BM, D))


def _dinv_call(degp):
  return pl.pallas_call(
      _dinv_body,
      grid=(N_PAD // BM,),
      in_specs=[pl.BlockSpec((NC, BM, D), lambda i: (0, i, 0))],
      out_specs=pl.BlockSpec((BM, D), lambda i: (i, 0)),
      out_shape=jax.ShapeDtypeStruct((N_PAD, D), jnp.float32),
  )(degp)


def _enc_body(x_ref, wn_ref, bn_ref, wc_ref, wr_ref, rb_ref, dv_ref,
              m_ref, r_ref):
  h = jnp.maximum(
      jnp.dot(x_ref[...], wn_ref[...], preferred_element_type=jnp.float32)
      + bn_ref[...], 0.0)
  m_ref[...] = jnp.dot(h, wc_ref[...],
                       preferred_element_type=jnp.float32) * dv_ref[...]
  r_ref[...] = jnp.dot(h, wr_ref[...],
                       preferred_element_type=jnp.float32) + rb_ref[...]


def _enc_call(x_pad, wn, bn, wc0, wr, rb, dinv):
  blk = pl.BlockSpec((BM, D), lambda i: (i, 0))
  wspec = pl.BlockSpec((D, D), lambda i: (0, 0))
  bspec = pl.BlockSpec((1, D), lambda i: (0, 0))
  return pl.pallas_call(
      _enc_body,
      grid=(N_PAD // BM,),
      in_specs=[blk, wspec, bspec, wspec, wspec, bspec, blk],
      out_specs=(blk, blk),
      out_shape=(jax.ShapeDtypeStruct((N_PAD, D), jnp.float32),
                 jax.ShapeDtypeStruct((N_PAD, D), jnp.float32)),
  )(x_pad, wn, bn, wc0, wr, rb, dinv)


def _layer_body(sp_ref, m_ref, r_ref, dv_ref, cb_ref, wc_ref, wr_ref, rb_ref,
                m2_ref, r2_ref):
  tot = sp_ref[0] + sp_ref[1] + m_ref[...]
  h = jnp.maximum(dv_ref[...] * tot + cb_ref[...] + r_ref[...], 0.0)
  m2_ref[...] = jnp.dot(h, wc_ref[...],
                        preferred_element_type=jnp.float32) * dv_ref[...]
  r2_ref[...] = jnp.dot(h, wr_ref[...],
                        preferred_element_type=jnp.float32) + rb_ref[...]


def _layer_call(sp, m, r, dinv, cb, wc_next, wr, rb):
  blk = pl.BlockSpec((BM, D), lambda i: (i, 0))
  spspec = pl.BlockSpec((NC, BM, D), lambda i: (0, i, 0))
  wspec = pl.BlockSpec((D, D), lambda i: (0, 0))
  bspec = pl.BlockSpec((1, D), lambda i: (0, 0))
  return pl.pallas_call(
      _layer_body,
      grid=(N_PAD // BM,),
      in_specs=[spspec, blk, blk, blk, bspec, wspec, wspec, bspec],
      out_specs=(blk, blk),
      out_shape=(jax.ShapeDtypeStruct((N_PAD, D), jnp.float32),
                 jax.ShapeDtypeStruct((N_PAD, D), jnp.float32)),
  )(sp, m, r, dinv, cb, wc_next, wr, rb)


def _final_body(sp_ref, m_ref, r_ref, dv_ref, cb_ref, wac_ref, ac_ref):
  tot = sp_ref[0] + sp_ref[1] + m_ref[...]
  h = jnp.maximum(dv_ref[...] * tot + cb_ref[...] + r_ref[...], 0.0)
  ac_ref[...] = jnp.dot(h, wac_ref[...], preferred_element_type=jnp.float32)


def _final_call(sp, m, r, dinv, cb, wac):
  blk = pl.BlockSpec((BM, D), lambda i: (i, 0))
  spspec = pl.BlockSpec((NC, BM, D), lambda i: (0, i, 0))
  bspec = pl.BlockSpec((1, D), lambda i: (0, 0))
  return pl.pallas_call(
      _final_body,
      grid=(N_PAD // BM,),
      in_specs=[spspec, blk, blk, blk, bspec,
                pl.BlockSpec((D, 2), lambda i: (0, 0))],
      out_specs=pl.BlockSpec((BM, 2), lambda i: (i, 0)),
      out_shape=jax.ShapeDtypeStruct((N_PAD, 2), jnp.float32),
  )(sp, m, r, dinv, cb, wac)


def _t_body(ea_ref, we_ref, be_ref, w3_ref, db_ref, t_ref):
  e = jnp.maximum(
      jnp.dot(ea_ref[...], we_ref[...], preferred_element_type=jnp.float32)
      + be_ref[...], 0.0)
  t_ref[...] = jnp.dot(e, w3_ref[...],
                       preferred_element_type=jnp.float32) + db_ref[...]


def _t_call(edge_attr, we, be, w3, db):
  return pl.pallas_call(
      _t_body,
      grid=(E // BE,),
      in_specs=[pl.BlockSpec((BE, DE), lambda i: (i, 0)),
                pl.BlockSpec((DE, D), lambda i: (0, 0)),
                pl.BlockSpec((1, D), lambda i: (0, 0)),
                pl.BlockSpec((D, 1), lambda i: (0, 0)),
                pl.BlockSpec((1, 1), lambda i: (0, 0))],
      out_specs=pl.BlockSpec((BE, 1), lambda i: (i, 0)),
      out_shape=jax.ShapeDtypeStruct((E, 1), jnp.float32),
  )(edge_attr, we, be, w3, db)


@jax.jit
def kernel(x, edge_index, edge_attr, enc_node_W, enc_node_b, enc_edge_W,
           enc_edge_b, conv_W, conv_b, res_W, res_b, dec_W, dec_b):
  ei = edge_index.astype(jnp.int32)
  row3 = jnp.pad(ei[0], (0, E_PAD - E)).reshape(NW, NCHUNK, CHUNK)
  col3 = jnp.pad(ei[1], (0, E_PAD - E),
                 constant_values=N).reshape(NW, NCHUNK, CHUNK)
  row_f = row3.reshape(GCHUNK_TOT, GC)
  col_f = col3.reshape(GCHUNK_TOT, GC)

  x_pad = jnp.pad(x, ((0, N_PAD - N), (0, 0)))
  bn = enc_node_b.reshape(1, D)
  be = enc_edge_b.reshape(1, D)
  rb = res_b.reshape(1, D)
  db = dec_b.reshape(1, 1)
  w3 = dec_W[2 * D:]
  wac = jnp.concatenate([dec_W[:D], dec_W[D:2 * D]], axis=1)

  degp = _hist(col3)
  dinv = _dinv_call(degp)

  t = _t_call(edge_attr, enc_edge_W, be, w3, db)
  t3 = jnp.pad(t[:, 0], (0, E_PAD - E)).reshape(NW, NCHUNK, CHUNK)

  m, r = _enc_call(x_pad, enc_node_W, bn, conv_W[0], res_W, rb, dinv)
  for i in range(NLAYER - 1):
    sp = _scatter(m, row_f, col_f)
    m, r = _layer_call(sp, m, r, dinv, conv_b[i].reshape(1, D),
                       conv_W[i + 1], res_W, rb)
  sp = _scatter(m, row_f, col_f)
  ac = _final_call(sp, m, r, dinv, conv_b[NLAYER - 1].reshape(1, D), wac)

  o3 = _edgeout(ac[:, 0], ac[:, 1], t3, row3, col3)
  return o3.reshape(E_PAD)[:E]


# speedup vs baseline: 1.0062x; 1.0062x over previous
import jax
import jax.numpy as jnp
from jax import lax
from jax.experimental import pallas as pl
from jax.experimental.pallas import tpu as pltpu
from jax.experimental.pallas import tpu_sc as plsc

N = 10000
E = 320000
D = 128
DE = 16
NLAYER = 3

NC, NS, LANES = 2, 16, 16
NW = NC * NS
N_PAD = 10240
CHUNK = 128
EPT = 10240
NCHUNK = EPT // CHUNK
E_PAD = EPT * NW
ROWS_PT = N_PAD // NS

_mesh = plsc.VectorSubcoreMesh(core_axis_name="c", subcore_axis_name="s")


def _hist_body(col_hbm, out_hbm, colv, ones_b, acc):
  c = lax.axis_index("c")
  s = lax.axis_index("s")
  wid = s * NC + c
  pltpu.sync_copy(col_hbm.at[wid], colv)
  onerow = jnp.where(jnp.arange(LANES, dtype=jnp.int32) == 0, 1.0, 0.0)
  zrow = jnp.zeros((LANES,), jnp.float32)

  @pl.loop(0, CHUNK)
  def _(i):
    for b in range(D // LANES):
      ones_b[i, pl.ds(b * LANES, LANES)] = zrow

  for p in range(ROWS_PT // CHUNK):
    pltpu.sync_copy(ones_b, acc.at[pl.ds(s * ROWS_PT + p * CHUNK, CHUNK)])

  @pl.loop(0, CHUNK)
  def _(i):
    ones_b[i, pl.ds(0, LANES)] = onerow

  plsc.subcore_barrier()

  @pl.loop(0, NCHUNK)
  def _(j):
    pltpu.sync_copy(ones_b, acc.at[colv.at[j]], add=True)

  plsc.subcore_barrier()
  for p in range(ROWS_PT // CHUNK):
    base = s * ROWS_PT + p * CHUNK
    pltpu.sync_copy(acc.at[pl.ds(base, CHUNK)], ones_b)
    pltpu.sync_copy(ones_b, out_hbm.at[c, pl.ds(base, CHUNK)])


_hist = pl.kernel(
    _hist_body,
    out_type=jax.ShapeDtypeStruct((NC, N_PAD, D), jnp.float32),
    mesh=_mesh,
    scratch_types=[
        pltpu.VMEM((NCHUNK, CHUNK), jnp.int32),
        pltpu.VMEM((CHUNK, D), jnp.float32),
        pltpu.VMEM_SHARED((N_PAD, D), jnp.float32),
    ],
)


GC = 128
QCS = 16
GCHUNK_TOT = E_PAD // GC
C0G = 144
C1G = (GCHUNK_TOT - 16 * C0G) // 16
GNBUF = 2
ZR = 64


def _scatter_body(m_hbm, row_hbm, col_hbm, out_hbm,
                  rowq, colq, g0, g1, zbuf, acc, s0, s1):
  c = lax.axis_index("c")
  s = lax.axis_index("s")
  bufs = (g0, g1)
  sems = (s0, s1)
  zrow = jnp.zeros((LANES,), jnp.float32)

  gstart = jnp.where(c == 0, s * C0G, 16 * C0G + s * C1G)
  nst = jnp.where(c == 0, C0G // QCS, C1G // QCS)

  pltpu.sync_copy(row_hbm.at[pl.ds(gstart, QCS)], rowq)
  pltpu.sync_copy(col_hbm.at[pl.ds(gstart, QCS)], colq)
  for b in range(GNBUF):
    pltpu.async_copy(m_hbm.at[rowq.at[b]], bufs[b], sems[b])

  @pl.loop(0, ZR)
  def _(i):
    for b in range(D // LANES):
      zbuf[i, pl.ds(b * LANES, LANES)] = zrow

  for p in range(ROWS_PT // ZR):
    pltpu.sync_copy(zbuf, acc.at[pl.ds(s * ROWS_PT + p * ZR, ZR)])
  plsc.subcore_barrier()

  @pl.loop(0, nst)
  def _(st):
    gb = gstart + st * QCS

    @pl.when(st > 0)
    def _():
      pltpu.sync_copy(row_hbm.at[pl.ds(gb, QCS)], rowq)
      pltpu.sync_copy(col_hbm.at[pl.ds(gb, QCS)], colq)
      for b in range(GNBUF):
        pltpu.async_copy(m_hbm.at[rowq.at[b]], bufs[b], sems[b])

    @pl.loop(0, QCS, step=GNBUF)
    def _(j):
      for b in range(GNBUF):
        k = j + b
        pltpu.make_async_copy(m_hbm.at[rowq.at[k]], bufs[b], sems[b]).wait()
        pltpu.sync_copy(bufs[b], acc.at[colq.at[k]], add=True)

        @pl.when(k + GNBUF < QCS)
        def _():
          pltpu.async_copy(m_hbm.at[rowq.at[k + GNBUF]], bufs[b], sems[b])

  plsc.subcore_barrier()
  for p in range(ROWS_PT // GC):
    base = s * ROWS_PT + p * GC
    pltpu.sync_copy(acc.at[pl.ds(base, GC)], out_hbm.at[c, pl.ds(base, GC)])


_scatter = pl.kernel(
    _scatter_body,
    out_type=jax.ShapeDtypeStruct((NC, N_PAD, D), jnp.float32),
    mesh=_mesh,
    scratch_types=[
        pltpu.VMEM((QCS, GC), jnp.int32),
        pltpu.VMEM((QCS, GC), jnp.int32),
        pltpu.VMEM((GC, D), jnp.float32),
        pltpu.VMEM((GC, D), jnp.float32),
        pltpu.VMEM((ZR, D), jnp.float32),
        pltpu.VMEM_SHARED((N_PAD, D), jnp.float32),
        pltpu.SemaphoreType.DMA,
        pltpu.SemaphoreType.DMA,
    ],
)


def _edgeout_body(a_hbm, c_hbm, t_hbm, row_hbm, col_hbm, out_hbm,
                  av, cv, tv, rowv, colv, obuf):
  c = lax.axis_index("c")
  s = lax.axis_index("s")
  wid = s * NC + c
  pltpu.sync_copy(a_hbm, av)
  pltpu.sync_copy(c_hbm, cv)
  pltpu.sync_copy(t_hbm.at[wid], tv)
  pltpu.sync_copy(row_hbm.at[wid], rowv)
  pltpu.sync_copy(col_hbm.at[wid], colv)

  @pl.loop(0, NCHUNK)
  def _(j):
    for b in range(CHUNK // LANES):
      sl = pl.ds(b * LANES, LANES)
      rv = rowv[j, sl]
      cc = colv[j, sl]
      o = plsc.load_gather(av, [rv]) + plsc.load_gather(cv, [cc]) + tv[j, sl]
      obuf[j, sl] = o

  pltpu.sync_copy(obuf, out_hbm.at[wid])


_edgeout = pl.kernel(
    _edgeout_body,
    out_type=jax.ShapeDtypeStruct((NW, NCHUNK, CHUNK), jnp.float32),
    mesh=_mesh,
    compiler_params=pltpu.CompilerParams(needs_layout_passes=False),
    scratch_types=[
        pltpu.VMEM((N_PAD,), jnp.float32),
        pltpu.VMEM((N_PAD,), jnp.float32),
        pltpu.VMEM((NCHUNK, CHUNK), jnp.float32),
        pltpu.VMEM((NCHUNK, CHUNK), jnp.int32),
        pltpu.VMEM((NCHUNK, CHUNK), jnp.int32),
        pltpu.VMEM((NCHUNK, CHUNK), jnp.float32),
    ],
)


BM = 1024
BE = 512


def _dinv_body(dp_ref, out_ref):
  deg = dp_ref[0, :, :1] + dp_ref[1, :, :1] + 1.0
  dv = lax.rsqrt(jnp.maximum(deg, 1e-12))
  out_ref[...] = jnp.broadcast_to(dv, (BM, D))


def _dinv_call(degp):
  return pl.pallas_call(
      _dinv_body,
      grid=(N_PAD // BM,),
      in_specs=[pl.BlockSpec((NC, BM, D), lambda i: (0, i, 0))],
      out_specs=pl.BlockSpec((BM, D), lambda i: (i, 0)),
      out_shape=jax.ShapeDtypeStruct((N_PAD, D), jnp.float32),
  )(degp)


def _enc_body(x_ref, wn_ref, bn_ref, wc_ref, wr_ref, rb_ref, dv_ref,
              m_ref, r_ref):
  h = jnp.maximum(
      jnp.dot(x_ref[...], wn_ref[...], preferred_element_type=jnp.float32)
      + bn_ref[...], 0.0)
  m_ref[...] = jnp.dot(h, wc_ref[...],
                       preferred_element_type=jnp.float32) * dv_ref[...]
  r_ref[...] = jnp.dot(h, wr_ref[...],
                       preferred_element_type=jnp.float32) + rb_ref[...]


def _enc_call(x_pad, wn, bn, wc0, wr, rb, dinv):
  blk = pl.BlockSpec((BM, D), lambda i: (i, 0))
  wspec = pl.BlockSpec((D, D), lambda i: (0, 0))
  bspec = pl.BlockSpec((1, D), lambda i: (0, 0))
  return pl.pallas_call(
      _enc_body,
      grid=(N_PAD // BM,),
      in_specs=[blk, wspec, bspec, wspec, wspec, bspec, blk],
      out_specs=(blk, blk),
      out_shape=(jax.ShapeDtypeStruct((N_PAD, D), jnp.float32),
                 jax.ShapeDtypeStruct((N_PAD, D), jnp.float32)),
  )(x_pad, wn, bn, wc0, wr, rb, dinv)


def _layer_body(sp_ref, m_ref, r_ref, dv_ref, cb_ref, wc_ref, wr_ref, rb_ref,
                m2_ref, r2_ref):
  tot = sp_ref[0] + sp_ref[1] + m_ref[...]
  h = jnp.maximum(dv_ref[...] * tot + cb_ref[...] + r_ref[...], 0.0)
  m2_ref[...] = jnp.dot(h, wc_ref[...],
                        preferred_element_type=jnp.float32) * dv_ref[...]
  r2_ref[...] = jnp.dot(h, wr_ref[...],
                        preferred_element_type=jnp.float32) + rb_ref[...]


def _layer_call(sp, m, r, dinv, cb, wc_next, wr, rb):
  blk = pl.BlockSpec((BM, D), lambda i: (i, 0))
  spspec = pl.BlockSpec((NC, BM, D), lambda i: (0, i, 0))
  wspec = pl.BlockSpec((D, D), lambda i: (0, 0))
  bspec = pl.BlockSpec((1, D), lambda i: (0, 0))
  return pl.pallas_call(
      _layer_body,
      grid=(N_PAD // BM,),
      in_specs=[spspec, blk, blk, blk, bspec, wspec, wspec, bspec],
      out_specs=(blk, blk),
      out_shape=(jax.ShapeDtypeStruct((N_PAD, D), jnp.float32),
                 jax.ShapeDtypeStruct((N_PAD, D), jnp.float32)),
  )(sp, m, r, dinv, cb, wc_next, wr, rb)


def _final_body(sp_ref, m_ref, r_ref, dv_ref, cb_ref, wac_ref, ac_ref):
  tot = sp_ref[0] + sp_ref[1] + m_ref[...]
  h = jnp.maximum(dv_ref[...] * tot + cb_ref[...] + r_ref[...], 0.0)
  ac_ref[...] = jnp.dot(h, wac_ref[...], preferred_element_type=jnp.float32)


def _final_call(sp, m, r, dinv, cb, wac):
  blk = pl.BlockSpec((BM, D), lambda i: (i, 0))
  spspec = pl.BlockSpec((NC, BM, D), lambda i: (0, i, 0))
  bspec = pl.BlockSpec((1, D), lambda i: (0, 0))
  return pl.pallas_call(
      _final_body,
      grid=(N_PAD // BM,),
      in_specs=[spspec, blk, blk, blk, bspec,
                pl.BlockSpec((D, 2), lambda i: (0, 0))],
      out_specs=pl.BlockSpec((BM, 2), lambda i: (i, 0)),
      out_shape=jax.ShapeDtypeStruct((N_PAD, 2), jnp.float32),
  )(sp, m, r, dinv, cb, wac)


def _t_body(ea_ref, we_ref, be_ref, w3_ref, db_ref, t_ref):
  e = jnp.maximum(
      jnp.dot(ea_ref[...], we_ref[...], preferred_element_type=jnp.float32)
      + be_ref[...], 0.0)
  t_ref[...] = jnp.dot(e, w3_ref[...],
                       preferred_element_type=jnp.float32) + db_ref[...]


def _t_call(edge_attr, we, be, w3, db):
  return pl.pallas_call(
      _t_body,
      grid=(E // BE,),
      in_specs=[pl.BlockSpec((BE, DE), lambda i: (i, 0)),
                pl.BlockSpec((DE, D), lambda i: (0, 0)),
                pl.BlockSpec((1, D), lambda i: (0, 0)),
                pl.BlockSpec((D, 1), lambda i: (0, 0)),
                pl.BlockSpec((1, 1), lambda i: (0, 0))],
      out_specs=pl.BlockSpec((BE, 1), lambda i: (i, 0)),
      out_shape=jax.ShapeDtypeStruct((E, 1), jnp.float32),
  )(edge_attr, we, be, w3, db)


@jax.jit
def kernel(x, edge_index, edge_attr, enc_node_W, enc_node_b, enc_edge_W,
           enc_edge_b, conv_W, conv_b, res_W, res_b, dec_W, dec_b):
  ei = edge_index.astype(jnp.int32)
  row3 = jnp.pad(ei[0], (0, E_PAD - E)).reshape(NW, NCHUNK, CHUNK)
  col3 = jnp.pad(ei[1], (0, E_PAD - E),
                 constant_values=N).reshape(NW, NCHUNK, CHUNK)
  row_f = row3.reshape(GCHUNK_TOT, GC)
  col_f = col3.reshape(GCHUNK_TOT, GC)

  x_pad = jnp.pad(x, ((0, N_PAD - N), (0, 0)))
  bn = enc_node_b.reshape(1, D)
  be = enc_edge_b.reshape(1, D)
  rb = res_b.reshape(1, D)
  db = dec_b.reshape(1, 1)
  w3 = dec_W[2 * D:]
  wac = jnp.concatenate([dec_W[:D], dec_W[D:2 * D]], axis=1)

  degp = _hist(col3)
  dinv = _dinv_call(degp)

  t = _t_call(edge_attr, enc_edge_W, be, w3, db)
  t3 = jnp.pad(t[:, 0], (0, E_PAD - E)).reshape(NW, NCHUNK, CHUNK)

  m, r = _enc_call(x_pad, enc_node_W, bn, conv_W[0], res_W, rb, dinv)
  for i in range(NLAYER - 1):
    sp = _scatter(m, row_f, col_f)
    m, r = _layer_call(sp, m, r, dinv, conv_b[i].reshape(1, D),
                       conv_W[i + 1], res_W, rb)
  sp = _scatter(m, row_f, col_f)
  ac = _final_call(sp, m, r, dinv, conv_b[NLAYER - 1].reshape(1, D), wac)

  o3 = _edgeout(ac[:, 0], ac[:, 1], t3, row3, col3)
  return o3.reshape(E_PAD)[:E]
